# baseline (speedup 1.0000x reference)
import jax
import jax.numpy as jnp
from jax.experimental import pallas as pl

N_CODES = 1024
D = 64
BETA = 0.25
TILE = 4096


def _vq_body(z_ref, cb_ref, cbm2_ref, e2_ref, out_ref, idx_ref, ls_ref):
    zb = z_ref[0]
    zt = zb.T
    cb = cb_ref[...]
    zem2 = jax.lax.dot_general(zt, cbm2_ref[...], (((1,), (1,)), ((), ())),
                               preferred_element_type=jnp.float32)
    z2 = jnp.sum(zt * zt, axis=1, keepdims=True)
    dists = (z2 + e2_ref[...]) + zem2
    m = jnp.min(dists, axis=1, keepdims=True)
    iota_f = jax.lax.broadcasted_iota(
        jnp.int32, (1, N_CODES), 1).astype(jnp.float32)
    idx_f = jnp.min(jnp.where(dists == m, iota_f, float(N_CODES)),
                    axis=1, keepdims=True)
    idx = idx_f[:, 0].astype(jnp.int32)
    oh = (iota_f == idx_f).astype(jnp.bfloat16)
    zqt = jax.lax.dot_general(cb, oh, (((0,), (1,)), ((), ())),
                              preferred_element_type=jnp.float32)
    diff = zqt - zb
    out_ref[0] = zb + diff
    idx_ref[0, 0, 0] = idx
    ls_ref[0, 0, 0] = jnp.sum(diff * diff, axis=0)


def kernel(z, codebook):
    B, Dc, T, H, W = z.shape
    S = T * H * W
    n_chunks = S // TILE
    z3 = z.reshape(B, Dc, S)
    cbm2 = -2.0 * codebook
    e2 = jnp.sum(codebook ** 2, axis=1)[None, :]
    grid = (B, n_chunks)
    zq3, idx4, lp = pl.pallas_call(
        _vq_body,
        grid=grid,
        in_specs=[
            pl.BlockSpec((1, Dc, TILE), lambda b, c: (b, 0, c)),
            pl.BlockSpec((N_CODES, Dc), lambda b, c: (0, 0)),
            pl.BlockSpec((N_CODES, Dc), lambda b, c: (0, 0)),
            pl.BlockSpec((1, N_CODES), lambda b, c: (0, 0)),
        ],
        out_specs=[
            pl.BlockSpec((1, Dc, TILE), lambda b, c: (b, 0, c)),
            pl.BlockSpec((1, 1, 1, TILE), lambda b, c: (b, c, 0, 0)),
            pl.BlockSpec((1, 1, 1, TILE), lambda b, c: (b, c, 0, 0)),
        ],
        out_shape=[
            jax.ShapeDtypeStruct((B, Dc, S), jnp.float32),
            jax.ShapeDtypeStruct((B, n_chunks, 1, TILE), jnp.int32),
            jax.ShapeDtypeStruct((B, n_chunks, 1, TILE), jnp.float32),
        ],
    )(z3, codebook, cbm2, e2)
    z_q_st = zq3.reshape(z.shape)
    idx = idx4.reshape(B, T, H, W)
    v = jnp.sum(lp) / (B * S * Dc)
    vq_loss = v + BETA * v
    return z_q_st, vq_loss, idx


# speedup vs baseline: 1.0019x; 1.0019x over previous
import jax
import jax.numpy as jnp
from jax.experimental import pallas as pl

N_CODES = 1024
D = 64
BETA = 0.25
TILE = 4096


def _vq_body(z_ref, cb_ref, cbm2_ref, e2_ref, out_ref, idx_ref, ls_ref):
    zb = z_ref[0]
    zt = zb.T
    cb = cb_ref[...]
    zem2 = jax.lax.dot_general(zt, cbm2_ref[...], (((1,), (1,)), ((), ())),
                               preferred_element_type=jnp.float32)
    z2 = jnp.sum(zt * zt, axis=1, keepdims=True)
    dists = (z2 + e2_ref[...]) + zem2
    m = jnp.min(dists, axis=1, keepdims=True)
    iota_f = jax.lax.broadcasted_iota(
        jnp.int32, (1, N_CODES), 1).astype(jnp.float32)
    idx_f = jnp.min(jnp.where(dists == m, iota_f, float(N_CODES)),
                    axis=1, keepdims=True)
    idx = idx_f[:, 0].astype(jnp.int32)
    oh = (iota_f == idx_f).astype(jnp.float32)
    zqt = jax.lax.dot_general(cb, oh, (((0,), (1,)), ((), ())),
                              preferred_element_type=jnp.float32)
    diff = zqt - zb
    out_ref[0] = zb + diff
    idx_ref[0, 0, 0] = idx
    ls_ref[0, 0, 0] = jnp.sum(diff * diff, axis=0)


def kernel(z, codebook):
    B, Dc, T, H, W = z.shape
    S = T * H * W
    n_chunks = S // TILE
    z3 = z.reshape(B, Dc, S)
    cbm2 = -2.0 * codebook
    e2 = jnp.sum(codebook ** 2, axis=1)[None, :]
    grid = (B, n_chunks)
    zq3, idx4, lp = pl.pallas_call(
        _vq_body,
        grid=grid,
        in_specs=[
            pl.BlockSpec((1, Dc, TILE), lambda b, c: (b, 0, c)),
            pl.BlockSpec((N_CODES, Dc), lambda b, c: (0, 0)),
            pl.BlockSpec((N_CODES, Dc), lambda b, c: (0, 0)),
            pl.BlockSpec((1, N_CODES), lambda b, c: (0, 0)),
        ],
        out_specs=[
            pl.BlockSpec((1, Dc, TILE), lambda b, c: (b, 0, c)),
            pl.BlockSpec((1, 1, 1, TILE), lambda b, c: (b, c, 0, 0)),
            pl.BlockSpec((1, 1, 1, TILE), lambda b, c: (b, c, 0, 0)),
        ],
        out_shape=[
            jax.ShapeDtypeStruct((B, Dc, S), jnp.float32),
            jax.ShapeDtypeStruct((B, n_chunks, 1, TILE), jnp.int32),
            jax.ShapeDtypeStruct((B, n_chunks, 1, TILE), jnp.float32),
        ],
    )(z3, codebook, cbm2, e2)
    z_q_st = zq3.reshape(z.shape)
    idx = idx4.reshape(B, T, H, W)
    v = jnp.sum(lp) / (B * S * Dc)
    vq_loss = v + BETA * v
    return z_q_st, vq_loss, idx
